# baseline (speedup 1.0000x reference)
import functools

import jax
import jax.numpy as jnp
from jax import lax
from jax.experimental import pallas as pl
from jax.experimental.pallas import tpu as pltpu
from jax.experimental.pallas import tpu_sc as plsc

NUM_EMB = 1_000_000
DIM = 32
BATCH = 16384

NUM_CORES = 2
NUM_SUBCORES = 16
NUM_WORKERS = NUM_CORES * NUM_SUBCORES
B_PER_W = BATCH // NUM_WORKERS
CHUNK = 128
NUM_CHUNKS = B_PER_W // CHUNK


@functools.partial(
    pl.kernel,
    mesh=plsc.VectorSubcoreMesh(core_axis_name="c", subcore_axis_name="s"),
    out_type=jax.ShapeDtypeStruct((BATCH, DIM), jnp.float32),
    scratch_types=[
        pltpu.VMEM((B_PER_W,), jnp.int32),
        pltpu.VMEM((B_PER_W, DIM), jnp.float32),
        pltpu.SemaphoreType.DMA,
    ],
    compiler_params=pltpu.CompilerParams(use_tc_tiling_on_sc=False),
)
def _gather_sc(idx_hbm, table_hbm, out_hbm, idx_v, rows_v, sem):
    wid = lax.axis_index("s") * NUM_CORES + lax.axis_index("c")
    base = wid * B_PER_W
    pltpu.sync_copy(idx_hbm.at[pl.ds(base, B_PER_W)], idx_v)
    copies = []
    for j in range(NUM_CHUNKS):
        copies.append(
            pltpu.async_copy(
                table_hbm.at[idx_v.at[pl.ds(j * CHUNK, CHUNK)]],
                rows_v.at[pl.ds(j * CHUNK, CHUNK)],
                sem,
            )
        )
    for c in copies:
        c.wait()
    pltpu.sync_copy(rows_v, out_hbm.at[pl.ds(base, B_PER_W)])


def kernel(indices, table):
    return _gather_sc(indices.astype(jnp.int32), table)


# speedup vs baseline: 1.6580x; 1.6580x over previous
import functools

import jax
import jax.numpy as jnp
from jax import lax
from jax.experimental import pallas as pl
from jax.experimental.pallas import tpu as pltpu
from jax.experimental.pallas import tpu_sc as plsc

NUM_EMB = 1_000_000
DIM = 32
BATCH = 16384

NUM_CORES = 2
NUM_SUBCORES = 16
NUM_LANES = 16
NUM_WORKERS = NUM_CORES * NUM_SUBCORES
B_PER_W = BATCH // NUM_WORKERS
NUM_CHUNKS = B_PER_W // NUM_LANES


@functools.partial(
    pl.kernel,
    mesh=plsc.VectorSubcoreMesh(core_axis_name="c", subcore_axis_name="s"),
    out_type=jax.ShapeDtypeStruct((BATCH, DIM), jnp.float32),
    scratch_types=[
        pltpu.VMEM((B_PER_W,), jnp.int32),
        pltpu.VMEM((B_PER_W, DIM), jnp.float32),
        pltpu.SemaphoreType.DMA,
    ],
    compiler_params=pltpu.CompilerParams(needs_layout_passes=False),
)
def _gather_sc(idx_hbm, table_hbm, out_hbm, idx_v, rows_v, sem):
    wid = lax.axis_index("s") * NUM_CORES + lax.axis_index("c")
    base = wid * B_PER_W

    pltpu.sync_copy(idx_hbm.at[pl.ds(base, B_PER_W)], idx_v)

    lane = lax.iota(jnp.int32, NUM_LANES)

    def chunk_body(j, carry):
        chunk = idx_v[pl.ds(j * NUM_LANES, NUM_LANES)]
        for k in range(NUM_LANES):
            r = jnp.max(jnp.where(lane == k, chunk, 0))
            pltpu.async_copy(
                table_hbm.at[pl.ds(r, 1)],
                rows_v.at[pl.ds(j * NUM_LANES + k, 1)],
                sem,
            )
        return carry

    lax.fori_loop(0, NUM_CHUNKS, chunk_body, 0)

    pltpu.make_async_copy(table_hbm.at[pl.ds(0, B_PER_W)], rows_v, sem).wait()

    pltpu.sync_copy(rows_v, out_hbm.at[pl.ds(base, B_PER_W)])


def kernel(indices, table):
    return _gather_sc(indices.astype(jnp.int32), table)


# speedup vs baseline: 3.8636x; 2.3303x over previous
import functools

import jax
import jax.numpy as jnp
from jax import lax
from jax.experimental import pallas as pl
from jax.experimental.pallas import tpu as pltpu
from jax.experimental.pallas import tpu_sc as plsc

NUM_EMB = 1_000_000
DIM = 32
BATCH = 16384

NUM_CORES = 2
NUM_SUBCORES = 16
NUM_LANES = 16
NUM_WORKERS = NUM_CORES * NUM_SUBCORES
B_PER_W = BATCH // NUM_WORKERS
SUB = 8
DIM_TILES = DIM // SUB
WIN = 128
ROUND = 16
NUM_ROUNDS = B_PER_W // ROUND


@functools.partial(
    pl.kernel,
    mesh=plsc.VectorSubcoreMesh(core_axis_name="c", subcore_axis_name="s"),
    out_type=jax.ShapeDtypeStruct((DIM_TILES, SUB, BATCH), jnp.float32),
    scratch_types=[
        pltpu.VMEM((B_PER_W,), jnp.int32),
        pltpu.VMEM((DIM_TILES, SUB, ROUND * WIN), jnp.float32),
        pltpu.VMEM((DIM_TILES, SUB, B_PER_W), jnp.float32),
        pltpu.SemaphoreType.DMA,
    ],
    compiler_params=pltpu.CompilerParams(needs_layout_passes=False),
)
def _gather_sc(idx_hbm, table_hbm, out_hbm, idx_v, buf, rows_v, sem):
    wid = lax.axis_index("s") * NUM_CORES + lax.axis_index("c")
    base = wid * B_PER_W

    pltpu.sync_copy(idx_hbm.at[pl.ds(base, B_PER_W)], idx_v)

    lane = lax.iota(jnp.int32, NUM_LANES)

    def round_body(g, carry):
        chunk = idx_v[pl.ds(g * ROUND, ROUND)]
        for k in range(ROUND):
            r = jnp.max(jnp.where(lane == k, chunk, 0))
            rq = pl.multiple_of((r >> 7) << 7, 128)
            pltpu.async_copy(
                table_hbm.at[:, :, pl.ds(rq, WIN)],
                buf.at[:, :, pl.ds(k * WIN, WIN)],
                sem,
            )
        pltpu.make_async_copy(
            table_hbm.at[:, :, pl.ds(0, ROUND * WIN)], buf, sem
        ).wait()

        pos = lane * WIN + (chunk & (WIN - 1))
        for a in range(DIM_TILES):
            a_vec = jnp.full((NUM_LANES,), a, jnp.int32)
            for s in range(SUB):
                s_vec = jnp.full((NUM_LANES,), s, jnp.int32)
                vals = plsc.load_gather(buf, [a_vec, s_vec, pos])
                rows_v[a, s, pl.ds(g * ROUND, ROUND)] = vals
        return carry

    lax.fori_loop(0, NUM_ROUNDS, round_body, 0)

    pltpu.sync_copy(rows_v, out_hbm.at[:, :, pl.ds(base, B_PER_W)])


def kernel(indices, table):
    tbl = table.T.reshape(DIM_TILES, SUB, NUM_EMB)
    out = _gather_sc(indices.astype(jnp.int32), tbl)
    return out.reshape(DIM, BATCH).T


# speedup vs baseline: 7.6083x; 1.9692x over previous
import functools

import jax
import jax.numpy as jnp
from jax import lax
from jax.experimental import pallas as pl
from jax.experimental.pallas import tpu as pltpu
from jax.experimental.pallas import tpu_sc as plsc

NUM_EMB = 1_000_000
DIM = 32
BATCH = 16384

NUM_CORES = 2
NUM_SUBCORES = 16
NUM_LANES = 16
NUM_WORKERS = NUM_CORES * NUM_SUBCORES
B_PER_W = BATCH // NUM_WORKERS
SUB = 8
DIM_TILES = DIM // SUB
WIN = 128
GRP = 16
ROUND = 64
NUM_ROUNDS = B_PER_W // ROUND
CHUNKS = ROUND // NUM_LANES


@functools.partial(
    pl.kernel,
    mesh=plsc.VectorSubcoreMesh(core_axis_name="c", subcore_axis_name="s"),
    out_type=jax.ShapeDtypeStruct((DIM_TILES, SUB, BATCH), jnp.float32),
    scratch_types=[
        pltpu.VMEM((B_PER_W,), jnp.int32),
        pltpu.VMEM((DIM_TILES, SUB, ROUND * GRP), jnp.float32),
        pltpu.VMEM((DIM_TILES, SUB, B_PER_W), jnp.float32),
        pltpu.SemaphoreType.DMA,
    ],
    compiler_params=pltpu.CompilerParams(needs_layout_passes=False),
)
def _gather_sc(idx_hbm, table_hbm, out_hbm, idx_v, buf, rows_v, sem):
    wid = lax.axis_index("s") * NUM_CORES + lax.axis_index("c")
    base = wid * B_PER_W

    pltpu.sync_copy(idx_hbm.at[pl.ds(base, B_PER_W)], idx_v)

    lane = lax.iota(jnp.int32, NUM_LANES)

    def round_body(g, carry):
        def row_body(i, carry2):
            chunk = idx_v[pl.ds(g * ROUND + ((i >> 4) << 4), NUM_LANES)]
            r = jnp.max(jnp.where(lane == (i & 15), chunk, 0))
            rq = pl.multiple_of((r >> 7) << 7, 128)
            m = (r >> 4) & 7

            def mk_branch(mm):
                def branch():
                    pltpu.async_copy(
                        table_hbm.at[:, :, pl.ds(rq, WIN)].at[
                            :, :, pl.ds(mm * GRP, GRP)
                        ],
                        buf.at[:, :, pl.ds(i * GRP, GRP)],
                        sem,
                    )

                return branch

            lax.switch(m, [mk_branch(mm) for mm in range(SUB)])
            return carry2

        lax.fori_loop(0, ROUND, row_body, 0)

        pltpu.make_async_copy(
            table_hbm.at[:, :, pl.ds(0, ROUND * GRP)], buf, sem
        ).wait()

        def ext_body(j, carry2):
            chunk = idx_v[pl.ds(g * ROUND + j * NUM_LANES, NUM_LANES)]
            pos = (j * NUM_LANES + lane) * GRP + (chunk & (GRP - 1))
            for a in range(DIM_TILES):
                a_vec = jnp.full((NUM_LANES,), a, jnp.int32)
                for s in range(SUB):
                    s_vec = jnp.full((NUM_LANES,), s, jnp.int32)
                    vals = plsc.load_gather(buf, [a_vec, s_vec, pos])
                    rows_v[
                        a, s, pl.ds(g * ROUND + j * NUM_LANES, NUM_LANES)
                    ] = vals
            return carry2

        lax.fori_loop(0, CHUNKS, ext_body, 0)
        return carry

    lax.fori_loop(0, NUM_ROUNDS, round_body, 0)

    pltpu.sync_copy(rows_v, out_hbm.at[:, :, pl.ds(base, B_PER_W)])


def kernel(indices, table):
    tbl = table.T.reshape(DIM_TILES, SUB, NUM_EMB)
    out = _gather_sc(indices.astype(jnp.int32), tbl)
    return out.reshape(DIM, BATCH).T
